# Initial kernel scaffold; baseline (speedup 1.0000x reference)
import jax
import jax.numpy as jnp
from jax.experimental import pallas as pl


def kernel(x, table):
    raise NotImplementedError("write your pallas kernel here")


# speedup vs baseline: 7.0327x; 7.0327x over previous
import functools

import jax
import jax.numpy as jnp
from jax import lax
from jax.experimental import pallas as pl
from jax.experimental.pallas import tpu as pltpu
from jax.experimental.pallas import tpu_sc as plsc

NC = 2
NS = 16
NW = NC * NS


def _gather_body(b_per_w, ch, d_model, table_hbm, idx_hbm, out_hbm,
                 idx_v, buf, sem):
    wid = lax.axis_index("s") * NC + lax.axis_index("c")
    base = wid * b_per_w
    pltpu.sync_copy(idx_hbm.at[pl.ds(base, b_per_w)], idx_v)
    n_chunks = b_per_w // ch

    def step(j, carry):
        off = j * ch
        pltpu.async_copy(table_hbm.at[idx_v.at[pl.ds(off, ch)]], buf,
                         sem).wait()
        pltpu.sync_copy(buf, out_hbm.at[pl.ds(base + off, ch)])
        return carry

    lax.fori_loop(0, n_chunks, step, 0)


@functools.cache
def _make_gather(v, d_model, b_total):
    assert b_total % (8 * NW) == 0
    b_per_w = b_total // NW
    ch = 32
    assert b_per_w % ch == 0 and ch <= 128
    mesh = plsc.VectorSubcoreMesh(core_axis_name="c", subcore_axis_name="s")
    return pl.kernel(
        functools.partial(_gather_body, b_per_w, ch, d_model),
        out_type=jax.ShapeDtypeStruct((b_total, d_model), jnp.float32),
        mesh=mesh,
        scratch_types=[
            pltpu.VMEM((b_per_w,), jnp.int32),
            pltpu.VMEM((ch, d_model), jnp.float32),
            pltpu.SemaphoreType.DMA,
        ],
    )


def kernel(x, table):
    b, s = x.shape
    v, d_model = table.shape
    idx = x.reshape(-1).astype(jnp.int32)
    out = _make_gather(v, d_model, b * s)(table, idx)
    return out.reshape(b, s, d_model)
